# baseline (speedup 1.0000x reference)
import functools
import jax
import jax.numpy as jnp
from jax import lax
from jax.experimental import pallas as pl
from jax.experimental.pallas import tpu as pltpu
from jax.experimental.pallas import tpu_sc as plsc

N = 10000
NP = 10240
E = 320000
D = 128
DH = 64
H = 256
C = 5
CP = 16

NC = 2
NS = 16
NW = NC * NS
K = 80
NB = 4
NB1 = 8
NCH1 = E // (NS * K)
NCH2 = E // (NW * K)
RPT = NP // NS


def _sc_agg1(xh, src3, dst3, zeros_init):
    mesh = plsc.VectorSubcoreMesh(core_axis_name="c", subcore_axis_name="s")

    @functools.partial(
        pl.kernel,
        mesh=mesh,
        out_type=jax.ShapeDtypeStruct((NC, NP, DH), jnp.float32),
        scratch_types=[
            pltpu.VMEM((NCH1 + NB1 - 1, K), jnp.int32),
            pltpu.VMEM((NCH1, K), jnp.int32),
            pltpu.VMEM((NB1 * K, DH), jnp.float32),
            pltpu.VMEM_SHARED((NP, DH), jnp.float32),
            pltpu.SemaphoreType.DMA(()),
            pltpu.SemaphoreType.DMA(()),
        ],
        compiler_params=pltpu.CompilerParams(use_tc_tiling_on_sc=False),
    )
    def k(xh_hbm, src_hbm, dst_hbm, init_hbm, out_hbm, src_v, dst_v, big,
          acc, gsem, ssem):
        cid = lax.axis_index("c")
        sid = lax.axis_index("s")

        def slot(i):
            return big.at[pl.ds((i % NB1) * K, K)]

        def g_desc(i):
            return pltpu.make_async_copy(xh_hbm.at[cid].at[src_v.at[i]],
                                         slot(i), gsem)

        def s_desc(i):
            return pltpu.make_async_copy(slot(i), acc.at[dst_v.at[i]], ssem)

        rows = pl.ds(sid * RPT, RPT)
        pltpu.sync_copy(init_hbm.at[rows], acc.at[rows])
        pltpu.sync_copy(src_hbm.at[sid], src_v)
        pltpu.sync_copy(dst_hbm.at[sid], dst_v)
        plsc.subcore_barrier()

        def fire(i, carry):
            g_desc(i).start()
            return carry

        lax.fori_loop(0, NB1 - 1, fire, 0)

        def body(i, carry):
            g_desc(i).wait()
            s_desc(i).start(add=True)

            @pl.when(i >= 1)
            def _():
                s_desc(i).wait()

            g_desc(i + NB1 - 1).start()
            return carry

        lax.fori_loop(0, NCH1, body, 0)
        s_desc(NCH1 - 1).wait()

        def drain(i, carry):
            g_desc(i).wait()
            return carry

        lax.fori_loop(NCH1, NCH1 + NB1 - 1, drain, 0)
        plsc.subcore_barrier()
        pltpu.sync_copy(acc.at[rows], out_hbm.at[cid].at[rows])

    return k(xh, src3, dst3, zeros_init)


def _sc_agg2(q, src3, dst3, init2):
    mesh = plsc.VectorSubcoreMesh(core_axis_name="c", subcore_axis_name="s")

    @functools.partial(
        pl.kernel,
        mesh=mesh,
        out_type=jax.ShapeDtypeStruct((NC, NP, CP), jnp.float32),
        scratch_types=[
            pltpu.VMEM((NCH2 + NB - 1, K), jnp.int32),
            pltpu.VMEM((NCH2, K), jnp.int32),
            pltpu.VMEM((NB * K, CP), jnp.float32),
            pltpu.VMEM_SHARED((NP, CP), jnp.float32),
            pltpu.VMEM_SHARED((NP, CP), jnp.float32),
            pltpu.SemaphoreType.DMA(()),
            pltpu.SemaphoreType.DMA(()),
        ],
        compiler_params=pltpu.CompilerParams(use_tc_tiling_on_sc=False),
    )
    def k(q_hbm, src_hbm, dst_hbm, init_hbm, out_hbm, src_v, dst_v, big, acc,
          qs, gsem, ssem):
        cid = lax.axis_index("c")
        sid = lax.axis_index("s")
        wid = sid * NC + cid

        def slot(i):
            return big.at[pl.ds((i % NB) * K, K)]

        def g_desc(i):
            return pltpu.make_async_copy(qs.at[src_v.at[i]], slot(i), gsem)

        def s_desc(i):
            return pltpu.make_async_copy(slot(i), acc.at[dst_v.at[i]], ssem)

        rows = pl.ds(sid * RPT, RPT)
        pltpu.sync_copy(q_hbm.at[rows], qs.at[rows])
        pltpu.sync_copy(init_hbm.at[cid].at[rows], acc.at[rows])
        pltpu.sync_copy(src_hbm.at[wid], src_v)
        pltpu.sync_copy(dst_hbm.at[wid], dst_v)
        plsc.subcore_barrier()

        def fire(i, carry):
            g_desc(i).start()
            return carry

        lax.fori_loop(0, NB - 1, fire, 0)

        def body(i, carry):
            g_desc(i).wait()
            s_desc(i).start(add=True)

            @pl.when(i >= 1)
            def _():
                s_desc(i).wait()

            g_desc(i + NB - 1).start()
            return carry

        lax.fori_loop(0, NCH2, body, 0)
        s_desc(NCH2 - 1).wait()

        def drain(i, carry):
            g_desc(i).wait()
            return carry

        lax.fori_loop(NCH2, NCH2 + NB - 1, drain, 0)
        plsc.subcore_barrier()
        pltpu.sync_copy(acc.at[rows], out_hbm.at[cid].at[rows])

    return k(q, src3, dst3, init2)


def _tc_mlp(partials, W1, b1, W2p):
    BN = 2048

    def body(p_ref, w1_ref, b1_ref, w2_ref, q_ref):
        a = jnp.concatenate([p_ref[0], p_ref[1]], axis=-1)
        h = jnp.dot(a, w1_ref[...], preferred_element_type=jnp.float32)
        h = jnp.maximum(h + b1_ref[...], 0.0)
        q_ref[...] = jnp.dot(h, w2_ref[...], preferred_element_type=jnp.float32)

    return pl.pallas_call(
        body,
        grid=(NP // BN,),
        in_specs=[
            pl.BlockSpec((NC, BN, DH), lambda i: (0, i, 0)),
            pl.BlockSpec((D, H), lambda i: (0, 0)),
            pl.BlockSpec((1, H), lambda i: (0, 0)),
            pl.BlockSpec((H, CP), lambda i: (0, 0)),
        ],
        out_specs=pl.BlockSpec((BN, CP), lambda i: (i, 0)),
        out_shape=jax.ShapeDtypeStruct((NP, CP), jnp.float32),
    )(partials, W1, b1, W2p)


def _tc_combine(partials2):
    def body(p_ref, o_ref):
        o_ref[...] = p_ref[0] + p_ref[1]

    return pl.pallas_call(
        body,
        in_specs=[pl.BlockSpec((NC, NP, CP), lambda: (0, 0, 0))],
        out_specs=pl.BlockSpec((NP, CP), lambda: (0, 0)),
        out_shape=jax.ShapeDtypeStruct((NP, CP), jnp.float32),
    )(partials2)


def _chunked(a, n_tiles, fill, extra):
    nch = E // (n_tiles * K)
    a = a.reshape(n_tiles, nch, K)
    if extra:
        pad = jnp.full((n_tiles, extra, K), fill, jnp.int32)
        a = jnp.concatenate([a, pad], axis=1)
    return a


def kernel(x, edge_index, W1, b1, W2, b2):
    src = edge_index[0]
    dst = edge_index[1]

    xp = jnp.pad(x, ((0, NP - N), (0, 0)))
    xh = jnp.stack([xp[:, :DH], xp[:, DH:]])
    partials = _sc_agg1(xh,
                        _chunked(src, NS, 0, NB1 - 1),
                        _chunked(dst, NS, 0, 0),
                        jnp.zeros((NP, DH), jnp.float32))

    W2p = jnp.pad(W2, ((0, 0), (0, CP - C)))
    q = _tc_mlp(partials, W1, b1.reshape(1, H), W2p)

    b2row = jnp.pad(b2, (0, CP - C))
    init2 = jnp.stack([jnp.broadcast_to(b2row, (NP, CP)),
                       jnp.zeros((NP, CP), jnp.float32)])
    partials2 = _sc_agg2(q,
                         _chunked(src, NW, 0, NB - 1),
                         _chunked(dst, NW, 0, 0),
                         init2)
    out = _tc_combine(partials2)
    return out[:N, :C]


# speedup vs baseline: 1.3184x; 1.3184x over previous
import functools
import jax
import jax.numpy as jnp
from jax import lax
from jax.experimental import pallas as pl
from jax.experimental.pallas import tpu as pltpu
from jax.experimental.pallas import tpu_sc as plsc

N = 10000
NP = 10240
E = 320000
D = 128
DH = 64
H = 256
C = 5
CP = 16

NC = 2
NS = 16
NW = NC * NS
K = 80
NB = 4
NCH1 = E // (NS * K)
NCH2 = E // (NW * K)
RPT = NP // NS


def _sc_agg1(xh, src3, dst3, zeros_init):
    mesh = plsc.VectorSubcoreMesh(core_axis_name="c", subcore_axis_name="s")

    @functools.partial(
        pl.kernel,
        mesh=mesh,
        out_type=jax.ShapeDtypeStruct((NC, NP, DH), jnp.float32),
        scratch_types=[
            pltpu.VMEM((NCH1 + NB - 1, K), jnp.int32),
            pltpu.VMEM((NCH1, K), jnp.int32),
            pltpu.VMEM((NB * K, DH), jnp.float32),
            pltpu.VMEM_SHARED((NP, DH), jnp.float32),
            pltpu.SemaphoreType.DMA(()),
            pltpu.SemaphoreType.DMA(()),
        ],
        compiler_params=pltpu.CompilerParams(use_tc_tiling_on_sc=False),
    )
    def k(xh_hbm, src_hbm, dst_hbm, init_hbm, out_hbm, src_v, dst_v, big,
          acc, gsem, ssem):
        cid = lax.axis_index("c")
        sid = lax.axis_index("s")

        def slot(i):
            return big.at[pl.ds((i % NB) * K, K)]

        def g_desc(i):
            return pltpu.make_async_copy(xh_hbm.at[cid].at[src_v.at[i]],
                                         slot(i), gsem)

        def s_desc(i):
            return pltpu.make_async_copy(slot(i), acc.at[dst_v.at[i]], ssem)

        rows = pl.ds(sid * RPT, RPT)
        pltpu.sync_copy(init_hbm.at[rows], acc.at[rows])
        pltpu.sync_copy(src_hbm.at[sid], src_v)
        pltpu.sync_copy(dst_hbm.at[sid], dst_v)
        plsc.subcore_barrier()

        def fire(i, carry):
            g_desc(i).start()
            return carry

        lax.fori_loop(0, NB - 1, fire, 0)

        def body(i, carry):
            g_desc(i).wait()
            s_desc(i).start(add=True)

            @pl.when(i >= 1)
            def _():
                s_desc(i).wait()

            g_desc(i + NB - 1).start()
            return carry

        lax.fori_loop(0, NCH1, body, 0)
        s_desc(NCH1 - 1).wait()

        def drain(i, carry):
            g_desc(i).wait()
            return carry

        lax.fori_loop(NCH1, NCH1 + NB - 1, drain, 0)
        plsc.subcore_barrier()
        pltpu.sync_copy(acc.at[rows], out_hbm.at[cid].at[rows])

    return k(xh, src3, dst3, zeros_init)


def _sc_agg2(q, src3, dst3, init2):
    mesh = plsc.VectorSubcoreMesh(core_axis_name="c", subcore_axis_name="s")

    @functools.partial(
        pl.kernel,
        mesh=mesh,
        out_type=jax.ShapeDtypeStruct((NC, NP, CP), jnp.float32),
        scratch_types=[
            pltpu.VMEM((NCH2 + NB - 1, K), jnp.int32),
            pltpu.VMEM((NCH2, K), jnp.int32),
            pltpu.VMEM((NB * K, CP), jnp.float32),
            pltpu.VMEM_SHARED((NP, CP), jnp.float32),
            pltpu.VMEM_SHARED((NP, CP), jnp.float32),
            pltpu.SemaphoreType.DMA(()),
            pltpu.SemaphoreType.DMA(()),
        ],
        compiler_params=pltpu.CompilerParams(use_tc_tiling_on_sc=False),
    )
    def k(q_hbm, src_hbm, dst_hbm, init_hbm, out_hbm, src_v, dst_v, big, acc,
          qs, gsem, ssem):
        cid = lax.axis_index("c")
        sid = lax.axis_index("s")
        wid = sid * NC + cid

        def slot(i):
            return big.at[pl.ds((i % NB) * K, K)]

        def g_desc(i):
            return pltpu.make_async_copy(qs.at[src_v.at[i]], slot(i), gsem)

        def s_desc(i):
            return pltpu.make_async_copy(slot(i), acc.at[dst_v.at[i]], ssem)

        rows = pl.ds(sid * RPT, RPT)
        pltpu.sync_copy(q_hbm.at[rows], qs.at[rows])
        pltpu.sync_copy(init_hbm.at[cid].at[rows], acc.at[rows])
        pltpu.sync_copy(src_hbm.at[wid], src_v)
        pltpu.sync_copy(dst_hbm.at[wid], dst_v)
        plsc.subcore_barrier()

        def fire(i, carry):
            g_desc(i).start()
            return carry

        lax.fori_loop(0, NB - 1, fire, 0)

        def body(i, carry):
            g_desc(i).wait()
            s_desc(i).start(add=True)

            @pl.when(i >= 1)
            def _():
                s_desc(i).wait()

            g_desc(i + NB - 1).start()
            return carry

        lax.fori_loop(0, NCH2, body, 0)
        s_desc(NCH2 - 1).wait()

        def drain(i, carry):
            g_desc(i).wait()
            return carry

        lax.fori_loop(NCH2, NCH2 + NB - 1, drain, 0)
        plsc.subcore_barrier()
        pltpu.sync_copy(acc.at[rows], out_hbm.at[cid].at[rows])

    return k(q, src3, dst3, init2)


def _tc_mlp(partials, W1, b1, W2p):
    BN = 2048

    def body(p_ref, w1_ref, b1_ref, w2_ref, q_ref):
        a = jnp.concatenate([p_ref[0], p_ref[1]], axis=-1)
        h = jnp.dot(a, w1_ref[...], preferred_element_type=jnp.float32)
        h = jnp.maximum(h + b1_ref[...], 0.0)
        q_ref[...] = jnp.dot(h, w2_ref[...], preferred_element_type=jnp.float32)

    return pl.pallas_call(
        body,
        grid=(NP // BN,),
        in_specs=[
            pl.BlockSpec((NC, BN, DH), lambda i: (0, i, 0)),
            pl.BlockSpec((D, H), lambda i: (0, 0)),
            pl.BlockSpec((1, H), lambda i: (0, 0)),
            pl.BlockSpec((H, CP), lambda i: (0, 0)),
        ],
        out_specs=pl.BlockSpec((BN, CP), lambda i: (i, 0)),
        out_shape=jax.ShapeDtypeStruct((NP, CP), jnp.float32),
    )(partials, W1, b1, W2p)


def _tc_combine(partials2):
    def body(p_ref, o_ref):
        o_ref[...] = p_ref[0] + p_ref[1]

    return pl.pallas_call(
        body,
        in_specs=[pl.BlockSpec((NC, NP, CP), lambda: (0, 0, 0))],
        out_specs=pl.BlockSpec((NP, CP), lambda: (0, 0)),
        out_shape=jax.ShapeDtypeStruct((NP, CP), jnp.float32),
    )(partials2)


def _chunked(a, n_tiles, fill, extra):
    nch = E // (n_tiles * K)
    a = a.reshape(n_tiles, nch, K)
    if extra:
        pad = jnp.full((n_tiles, extra, K), fill, jnp.int32)
        a = jnp.concatenate([a, pad], axis=1)
    return a


def kernel(x, edge_index, W1, b1, W2, b2):
    src = edge_index[0]
    dst = edge_index[1]

    xp = jnp.pad(x, ((0, NP - N), (0, 0)))
    xh = jnp.stack([xp[:, :DH], xp[:, DH:]])
    partials = _sc_agg1(xh,
                        _chunked(src, NS, 0, NB - 1),
                        _chunked(dst, NS, 0, 0),
                        jnp.zeros((NP, DH), jnp.float32))

    W2p = jnp.pad(W2, ((0, 0), (0, CP - C)))
    q = _tc_mlp(partials, W1, b1.reshape(1, H), W2p)

    b2row = jnp.pad(b2, (0, CP - C))
    init2 = jnp.stack([jnp.broadcast_to(b2row, (NP, CP)),
                       jnp.zeros((NP, CP), jnp.float32)])
    partials2 = _sc_agg2(q,
                         _chunked(src, NW, 0, NB - 1),
                         _chunked(dst, NW, 0, 0),
                         init2)
    out = _tc_combine(partials2)
    return out[:N, :C]
